# baseline (speedup 1.0000x reference)
import jax
import jax.numpy as jnp
from jax import lax
from jax.experimental import pallas as pl
from jax.experimental.pallas import tpu as pltpu
from jax.experimental.pallas import tpu_sc as plsc

_NB_CLASSES = 1000000
_EMBED_DIM = 101
_BATCH = 16384

_NUM_CORES = 2
_NUM_SUBCORES = 16
_NUM_WORKERS = _NUM_CORES * _NUM_SUBCORES
_B_PER_W = _BATCH // _NUM_WORKERS


def _gather_body(idx_hbm, table_hbm, out_hbm, idx_v, rows_v, sem_in, sem_out):
  wid = lax.axis_index("s") * _NUM_CORES + lax.axis_index("c")
  base = wid * _B_PER_W
  pltpu.sync_copy(idx_hbm.at[pl.ds(base, _B_PER_W)], idx_v)

  def fire(q, carry):
    vec = idx_v[pl.ds(q * 16, 16)]
    for t in range(16):
      r = q * 16 + t
      pltpu.async_copy(table_hbm.at[pl.ds(vec[t], 1)],
                       rows_v.at[pl.ds(r, 1)], sem_in)
    return carry

  lax.fori_loop(0, _B_PER_W // 16, fire, 0)
  pltpu.make_async_copy(table_hbm.at[pl.ds(0, _B_PER_W)], rows_v,
                        sem_in).wait()
  pltpu.async_copy(rows_v, out_hbm.at[pl.ds(base, _B_PER_W)], sem_out).wait()


_TBLK = 2048


def _transpose_body(tab_t_ref, out_ref):
  out_ref[...] = tab_t_ref[...].T


def _retile(cls_table):
  return pl.pallas_call(
      _transpose_body,
      grid=(_NB_CLASSES // _TBLK,),
      in_specs=[pl.BlockSpec((_EMBED_DIM, _TBLK), lambda i: (0, i))],
      out_specs=pl.BlockSpec((_TBLK, _EMBED_DIM), lambda i: (i, 0)),
      out_shape=jax.ShapeDtypeStruct((_NB_CLASSES, _EMBED_DIM), jnp.float32),
  )(cls_table.T)


@jax.jit
def _gather(indices, cls_table):
  mesh = plsc.VectorSubcoreMesh(core_axis_name="c", subcore_axis_name="s")
  return pl.kernel(
      _gather_body,
      out_type=jax.ShapeDtypeStruct((_BATCH, _EMBED_DIM), jnp.float32),
      mesh=mesh,
      compiler_params=pltpu.CompilerParams(skip_device_barrier=True),
      scratch_types=[
          pltpu.VMEM((_B_PER_W,), jnp.int32),
          pltpu.VMEM((_B_PER_W, _EMBED_DIM), jnp.float32),
          pltpu.SemaphoreType.DMA,
          pltpu.SemaphoreType.DMA,
      ],
  )(indices, _retile(cls_table))


def kernel(indices, cls_table):
  return _gather(indices, cls_table)


# speedup vs baseline: 1.3470x; 1.3470x over previous
import jax
import jax.numpy as jnp
from jax import lax
from jax.experimental import pallas as pl
from jax.experimental.pallas import tpu as pltpu
from jax.experimental.pallas import tpu_sc as plsc

_NB_CLASSES = 1000000
_EMBED_DIM = 101
_BATCH = 16384

_NUM_CORES = 2
_NUM_SUBCORES = 16
_NUM_WORKERS = _NUM_CORES * _NUM_SUBCORES
_B_PER_W = _BATCH // _NUM_WORKERS


def _gather_body(idx_hbm, table_hbm, out_hbm, idx_v, rows_v, sem_in, sem_out):
  wid = lax.axis_index("s") * _NUM_CORES + lax.axis_index("c")
  base = wid * _B_PER_W
  pltpu.sync_copy(idx_hbm.at[pl.ds(base, _B_PER_W)], idx_v)

  def fire(q, carry):
    vec = idx_v[pl.ds(q * 16, 16)]
    for t in range(16):
      r = q * 16 + t
      pltpu.async_copy(table_hbm.at[pl.ds(vec[t], 1)],
                       rows_v.at[pl.ds(r, 1)], sem_in)
    return carry

  lax.fori_loop(0, _B_PER_W // 16, fire, 0)
  pltpu.make_async_copy(table_hbm.at[pl.ds(0, _B_PER_W)], rows_v,
                        sem_in).wait()
  pltpu.async_copy(rows_v, out_hbm.at[pl.ds(base, _B_PER_W)], sem_out).wait()


_TBLK = 4096


def _transpose_body(tab_t_ref, out_ref):
  out_ref[...] = tab_t_ref[...].T


def _retile(cls_table):
  return pl.pallas_call(
      _transpose_body,
      grid=((_NB_CLASSES + _TBLK - 1) // _TBLK,),
      in_specs=[pl.BlockSpec((_EMBED_DIM, _TBLK), lambda i: (0, i))],
      out_specs=pl.BlockSpec((_TBLK, _EMBED_DIM), lambda i: (i, 0)),
      out_shape=jax.ShapeDtypeStruct((_NB_CLASSES, _EMBED_DIM), jnp.float32),
  )(cls_table.T)


@jax.jit
def _gather(indices, cls_table):
  mesh = plsc.VectorSubcoreMesh(core_axis_name="c", subcore_axis_name="s")
  return pl.kernel(
      _gather_body,
      out_type=jax.ShapeDtypeStruct((_BATCH, _EMBED_DIM), jnp.float32),
      mesh=mesh,
      compiler_params=pltpu.CompilerParams(skip_device_barrier=True),
      scratch_types=[
          pltpu.VMEM((_B_PER_W,), jnp.int32),
          pltpu.VMEM((_B_PER_W, _EMBED_DIM), jnp.float32),
          pltpu.SemaphoreType.DMA,
          pltpu.SemaphoreType.DMA,
      ],
  )(indices, _retile(cls_table))


def kernel(indices, cls_table):
  return _gather(indices, cls_table)
